# baseline (speedup 1.0000x reference)
import functools

import jax
import jax.numpy as jnp
from jax import lax
from jax.experimental import pallas as pl
from jax.experimental.pallas import tpu as pltpu


def _round_up(v, m):
    return ((v + m - 1) // m) * m


def _pick_time_chunk(T):
    if T <= 64:
        return T
    for tt in (64, 32, 16, 8):
        if T % tt == 0:
            return tt
    return T


def _gru_body(x_ref, h0_ref, wih0_ref, whh0_ref, bih0_ref, bhh0_ref,
              wih1_ref, whh1_ref, bih1_ref, bhh1_ref, wfc_ref, bfc_ref,
              out_ref, hout_ref, seq_ref, gx0_ref, gx1_ref,
              wi0_ref, wh0_ref, wi1_ref, wh1_ref,
              bx0_ref, bh0_ref, bx1_ref, bh1_ref,
              *, Tt, H, nc, unroll):
    c = pl.program_id(1)
    NB = h0_ref.shape[1]

    @pl.when(c == 0)
    def _():
        hout_ref[...] = h0_ref[...]
        for dst, src in ((wi0_ref, wih0_ref), (wh0_ref, whh0_ref),
                         (wi1_ref, wih1_ref), (wh1_ref, whh1_ref)):
            dst[:, 0 * H:1 * H] = src[0]
            dst[:, 1 * H:2 * H] = src[1]
            dst[:, 2 * H:3 * H] = src[2]
        for bx, bh, bi, bb in ((bx0_ref, bh0_ref, bih0_ref, bhh0_ref),
                               (bx1_ref, bh1_ref, bih1_ref, bhh1_ref)):
            bx[:, 0 * H:1 * H] = bi[0] + bb[0]
            bx[:, 1 * H:2 * H] = bi[1] + bb[1]
            bx[:, 2 * H:3 * H] = bi[2]
            bh[:, 0 * H:2 * H] = jnp.zeros((1, 2 * H), jnp.float32)
            bh[:, 2 * H:3 * H] = bb[2]

    @pl.when(c > 0)
    def _():
        gx1_ref[...] = (jnp.dot(seq_ref[...], wi1_ref[...],
                                preferred_element_type=jnp.float32)
                        + bx1_ref[...])

    @pl.when(c < nc)
    def _():
        xin = x_ref[...].reshape(Tt * NB, x_ref.shape[-1])
        gx0_ref[...] = (jnp.dot(xin, wi0_ref[...],
                                preferred_element_type=jnp.float32)
                        + bx0_ref[...])

    w0hh = wh0_ref[...]
    b0hh = bh0_ref[...]
    w1hh = wh1_ref[...]
    b1hh = bh1_ref[...]

    def step0(t, h0):
        row = t * NB
        gh = jnp.dot(h0, w0hh, preferred_element_type=jnp.float32) + b0hh
        gx = gx0_ref[pl.ds(row, NB), :]
        r = jax.nn.sigmoid(gx[:, :H] + gh[:, :H])
        z = jax.nn.sigmoid(gx[:, H:2 * H] + gh[:, H:2 * H])
        n = jnp.tanh(gx[:, 2 * H:] + r * gh[:, 2 * H:])
        h_new = (1.0 - z) * n + z * h0
        seq_ref[pl.ds(row, NB), :] = h_new
        return h_new

    def step1(t, h1):
        row = t * NB
        gh = jnp.dot(h1, w1hh, preferred_element_type=jnp.float32) + b1hh
        gx = gx1_ref[pl.ds(row, NB), :]
        r = jax.nn.sigmoid(gx[:, :H] + gh[:, :H])
        z = jax.nn.sigmoid(gx[:, H:2 * H] + gh[:, H:2 * H])
        n = jnp.tanh(gx[:, 2 * H:] + r * gh[:, 2 * H:])
        return (1.0 - z) * n + z * h1

    def step_both(t, carry):
        h0, h1 = carry
        return step0(t, h0), step1(t, h1)

    @pl.when(c == 0)
    def _():
        hout_ref[0] = lax.fori_loop(0, Tt, step0, hout_ref[0], unroll=unroll)

    @pl.when(jnp.logical_and(c > 0, c < nc))
    def _():
        h0f, h1f = lax.fori_loop(0, Tt, step_both,
                                 (hout_ref[0], hout_ref[1]), unroll=unroll)
        hout_ref[0] = h0f
        hout_ref[1] = h1f

    @pl.when(c == nc)
    def _():
        h1f = lax.fori_loop(0, Tt, step1, hout_ref[1], unroll=unroll)
        hout_ref[1] = h1f
        h_last = jnp.maximum(h1f, 0.0)
        out_ref[...] = (jnp.dot(h_last, wfc_ref[...],
                                preferred_element_type=jnp.float32)
                        + bfc_ref[...])


@jax.jit
def _forward(x, h, fc_w, fc_b,
             l0_wih, l0_whh, l0_bih, l0_bhh,
             l1_wih, l1_whh, l1_bih, l1_bhh):
    B, D, T = x.shape
    L, _, H = h.shape
    C = fc_w.shape[1]

    x_btd = jnp.reshape(x, (B, T, D))

    Bp = _round_up(B, 16)
    if Bp != B:
        x_btd = jnp.pad(x_btd, ((0, Bp - B), (0, 0), (0, 0)))
        h = jnp.pad(h, ((0, 0), (0, Bp - B), (0, 0)))
    NB = Bp // 2

    x4d = jnp.reshape(jnp.transpose(x_btd, (1, 0, 2)), (T, 2, NB, D))

    Tt = _pick_time_chunk(T)
    nc = T // Tt
    unroll = True if Tt <= 16 else (4 if Tt % 4 == 0 else 1)

    def const3(b, c):
        return (0, 0, 0)

    in_specs = [
        pl.BlockSpec((Tt, 1, NB, D),
                     lambda b, c: (jnp.minimum(c, nc - 1), b, 0, 0)),
        pl.BlockSpec((L, NB, H), lambda b, c: (0, b, 0)),
        pl.BlockSpec((3, D, H), const3),
        pl.BlockSpec((3, H, H), const3),
        pl.BlockSpec((3, 1, H), const3),
        pl.BlockSpec((3, 1, H), const3),
        pl.BlockSpec((3, H, H), const3),
        pl.BlockSpec((3, H, H), const3),
        pl.BlockSpec((3, 1, H), const3),
        pl.BlockSpec((3, 1, H), const3),
        pl.BlockSpec((H, C), lambda b, c: (0, 0)),
        pl.BlockSpec((1, C), lambda b, c: (0, 0)),
    ]
    out_specs = [
        pl.BlockSpec((NB, C), lambda b, c: (b, 0)),
        pl.BlockSpec((L, NB, H), lambda b, c: (0, b, 0)),
    ]

    out, h_new = pl.pallas_call(
        functools.partial(_gru_body, Tt=Tt, H=H, nc=nc, unroll=unroll),
        out_shape=(jax.ShapeDtypeStruct((Bp, C), jnp.float32),
                   jax.ShapeDtypeStruct((L, Bp, H), jnp.float32)),
        grid_spec=pltpu.PrefetchScalarGridSpec(
            num_scalar_prefetch=0,
            grid=(2, nc + 1),
            in_specs=in_specs,
            out_specs=out_specs,
            scratch_shapes=[pltpu.VMEM((Tt * NB, H), jnp.float32),
                            pltpu.VMEM((Tt * NB, 3 * H), jnp.float32),
                            pltpu.VMEM((Tt * NB, 3 * H), jnp.float32),
                            pltpu.VMEM((D, 3 * H), jnp.float32),
                            pltpu.VMEM((H, 3 * H), jnp.float32),
                            pltpu.VMEM((H, 3 * H), jnp.float32),
                            pltpu.VMEM((H, 3 * H), jnp.float32),
                            pltpu.VMEM((1, 3 * H), jnp.float32),
                            pltpu.VMEM((1, 3 * H), jnp.float32),
                            pltpu.VMEM((1, 3 * H), jnp.float32),
                            pltpu.VMEM((1, 3 * H), jnp.float32)],
        ),
        compiler_params=pltpu.CompilerParams(
            dimension_semantics=("parallel", "arbitrary")),
    )(x4d, h, l0_wih, l0_whh, l0_bih, l0_bhh,
      l1_wih, l1_whh, l1_bih, l1_bhh, fc_w, fc_b)

    return out[:B], h_new[:, :B]


def kernel(x, h, fc_w, fc_b,
           layer0_wih, layer0_whh, layer0_bih, layer0_bhh,
           layer1_wih, layer1_whh, layer1_bih, layer1_bhh):
    return _forward(x, h, fc_w, fc_b,
                    layer0_wih, layer0_whh, layer0_bih, layer0_bhh,
                    layer1_wih, layer1_whh, layer1_bih, layer1_bhh)


# speedup vs baseline: 1.0127x; 1.0127x over previous
import functools

import jax
import jax.numpy as jnp
from jax import lax
from jax.experimental import pallas as pl
from jax.experimental.pallas import tpu as pltpu


def _round_up(v, m):
    return ((v + m - 1) // m) * m


def _pick_time_chunk(T):
    if T <= 64:
        return T
    for tt in (64, 32, 16, 8):
        if T % tt == 0:
            return tt
    return T


def _gru_body(x_ref, h0_ref, wih0_ref, whh0_ref, bih0_ref, bhh0_ref,
              wih1_ref, whh1_ref, bih1_ref, bhh1_ref, wfc_ref, bfc_ref,
              out_ref, hout_ref, seq_ref, gx0_ref, gx1_ref,
              *, Tt, H, nc, unroll):
    c = pl.program_id(1)
    NB = h0_ref.shape[1]
    D = x_ref.shape[-1]

    @pl.when(c == 0)
    def _():
        hout_ref[...] = h0_ref[...]

    @pl.when(c > 0)
    def _():
        gx1_ref[...] = (jnp.dot(seq_ref[...], wih1_ref[...],
                                preferred_element_type=jnp.float32)
                        + bih1_ref[...])

    @pl.when(c < nc)
    def _():
        xin = x_ref[...].reshape(Tt * NB, D)
        gx0_ref[...] = (jnp.dot(xin, wih0_ref[...],
                                preferred_element_type=jnp.float32)
                        + bih0_ref[...])

    w0hh = whh0_ref[...]
    b0hh = bhh0_ref[...]
    w1hh = whh1_ref[...]
    b1hh = bhh1_ref[...]

    def step0(t, h0):
        row = t * NB
        gh = jnp.dot(h0, w0hh, preferred_element_type=jnp.float32) + b0hh
        gx = gx0_ref[pl.ds(row, NB), :]
        r = jax.nn.sigmoid(gx[:, :H] + gh[:, :H])
        z = jax.nn.sigmoid(gx[:, H:2 * H] + gh[:, H:2 * H])
        n = jnp.tanh(gx[:, 2 * H:] + r * gh[:, 2 * H:])
        h_new = (1.0 - z) * n + z * h0
        seq_ref[pl.ds(row, NB), :] = h_new
        return h_new

    def step1(t, h1):
        row = t * NB
        gh = jnp.dot(h1, w1hh, preferred_element_type=jnp.float32) + b1hh
        gx = gx1_ref[pl.ds(row, NB), :]
        r = jax.nn.sigmoid(gx[:, :H] + gh[:, :H])
        z = jax.nn.sigmoid(gx[:, H:2 * H] + gh[:, H:2 * H])
        n = jnp.tanh(gx[:, 2 * H:] + r * gh[:, 2 * H:])
        return (1.0 - z) * n + z * h1

    def step_both(t, carry):
        h0, h1 = carry
        return step0(t, h0), step1(t, h1)

    @pl.when(c == 0)
    def _():
        hout_ref[0] = lax.fori_loop(0, Tt, step0, hout_ref[0], unroll=unroll)

    @pl.when(jnp.logical_and(c > 0, c < nc))
    def _():
        h0f, h1f = lax.fori_loop(0, Tt, step_both,
                                 (hout_ref[0], hout_ref[1]), unroll=unroll)
        hout_ref[0] = h0f
        hout_ref[1] = h1f

    @pl.when(c == nc)
    def _():
        h1f = lax.fori_loop(0, Tt, step1, hout_ref[1], unroll=unroll)
        hout_ref[1] = h1f
        h_last = jnp.maximum(h1f, 0.0)
        out_ref[...] = (jnp.dot(h_last, wfc_ref[...],
                                preferred_element_type=jnp.float32)
                        + bfc_ref[...])


def _cat_w(w):
    return jnp.concatenate([w[0], w[1], w[2]], axis=1)


def _cat_b(b):
    return jnp.concatenate([b[0], b[1], b[2]], axis=1)


@jax.jit
def _forward(x, h, fc_w, fc_b,
             l0_wih, l0_whh, l0_bih, l0_bhh,
             l1_wih, l1_whh, l1_bih, l1_bhh):
    B, D, T = x.shape
    L, _, H = h.shape
    C = fc_w.shape[1]

    x_btd = jnp.reshape(x, (B, T, D))

    Bp = _round_up(B, 16)
    if Bp != B:
        x_btd = jnp.pad(x_btd, ((0, Bp - B), (0, 0), (0, 0)))
        h = jnp.pad(h, ((0, 0), (0, Bp - B), (0, 0)))
    NB = Bp // 2

    x_t = jnp.transpose(x_btd, (1, 0, 2))

    Tt = _pick_time_chunk(T)
    nc = T // Tt
    unroll = True if Tt <= 16 else (4 if Tt % 4 == 0 else 1)

    w0ih, w1ih = _cat_w(l0_wih), _cat_w(l1_wih)
    w0hh, w1hh = _cat_w(l0_whh), _cat_w(l1_whh)
    b0ih, b1ih = _cat_b(l0_bih), _cat_b(l1_bih)
    b0hh, b1hh = _cat_b(l0_bhh), _cat_b(l1_bhh)

    def const2(b, c):
        return (0, 0)

    in_specs = [
        pl.BlockSpec((Tt, NB, D),
                     lambda b, c: (jnp.minimum(c, nc - 1), b, 0)),
        pl.BlockSpec((L, NB, H), lambda b, c: (0, b, 0)),
        pl.BlockSpec((D, 3 * H), const2),
        pl.BlockSpec((H, 3 * H), const2),
        pl.BlockSpec((1, 3 * H), const2),
        pl.BlockSpec((1, 3 * H), const2),
        pl.BlockSpec((H, 3 * H), const2),
        pl.BlockSpec((H, 3 * H), const2),
        pl.BlockSpec((1, 3 * H), const2),
        pl.BlockSpec((1, 3 * H), const2),
        pl.BlockSpec((H, C), const2),
        pl.BlockSpec((1, C), const2),
    ]
    out_specs = [
        pl.BlockSpec((NB, C), lambda b, c: (b, 0)),
        pl.BlockSpec((L, NB, H), lambda b, c: (0, b, 0)),
    ]

    out, h_new = pl.pallas_call(
        functools.partial(_gru_body, Tt=Tt, H=H, nc=nc, unroll=unroll),
        out_shape=(jax.ShapeDtypeStruct((Bp, C), jnp.float32),
                   jax.ShapeDtypeStruct((L, Bp, H), jnp.float32)),
        grid_spec=pltpu.PrefetchScalarGridSpec(
            num_scalar_prefetch=0,
            grid=(2, nc + 1),
            in_specs=in_specs,
            out_specs=out_specs,
            scratch_shapes=[pltpu.VMEM((Tt * NB, H), jnp.float32),
                            pltpu.VMEM((Tt * NB, 3 * H), jnp.float32),
                            pltpu.VMEM((Tt * NB, 3 * H), jnp.float32)],
        ),
        compiler_params=pltpu.CompilerParams(
            dimension_semantics=("parallel", "arbitrary")),
    )(x_t, h, w0ih, w0hh, b0ih, b0hh, w1ih, w1hh, b1ih, b1hh, fc_w, fc_b)

    return out[:B], h_new[:, :B]


def kernel(x, h, fc_w, fc_b,
           layer0_wih, layer0_whh, layer0_bih, layer0_bhh,
           layer1_wih, layer1_whh, layer1_bih, layer1_bhh):
    return _forward(x, h, fc_w, fc_b,
                    layer0_wih, layer0_whh, layer0_bih, layer0_bhh,
                    layer1_wih, layer1_whh, layer1_bih, layer1_bhh)


# speedup vs baseline: 1.0553x; 1.0420x over previous
import functools

import jax
import jax.numpy as jnp
from jax import lax
from jax.experimental import pallas as pl
from jax.experimental.pallas import tpu as pltpu


def _round_up(v, m):
    return ((v + m - 1) // m) * m


def _pick_time_chunk(T):
    if T <= 64:
        return T
    for tt in (64, 32, 16, 8):
        if T % tt == 0:
            return tt
    return T


def _gru_body(x_ref, h0_ref, wih0_ref, whh0_ref, bih0_ref, bhh0_ref,
              wih1_ref, whh1_ref, bih1_ref, bhh1_ref, wfc_ref, bfc_ref,
              out_ref, hout_ref, seq_ref, gx0_ref, gx1_ref,
              *, Tt, H, nc, unroll):
    c = pl.program_id(1)
    NB = h0_ref.shape[1]
    D = x_ref.shape[-1]

    @pl.when(c == 0)
    def _():
        hout_ref[...] = h0_ref[...]

    @pl.when(c > 0)
    def _():
        gx1_ref[...] = (jnp.dot(seq_ref[...], wih1_ref[...],
                                preferred_element_type=jnp.float32)
                        + bih1_ref[...])

    @pl.when(c < nc)
    def _():
        xin = x_ref[...].reshape(Tt * NB, D)
        gx0_ref[...] = (jnp.dot(xin, wih0_ref[...],
                                preferred_element_type=jnp.float32)
                        + bih0_ref[...])

    w0hh = whh0_ref[...]
    b0hh = bhh0_ref[...]
    w1hh = whh1_ref[...]
    b1hh = bhh1_ref[...]

    def step0(t, h0):
        row = t * NB
        gh = jnp.dot(h0, w0hh, preferred_element_type=jnp.float32) + b0hh
        gx = gx0_ref[pl.ds(row, NB), :]
        r = jax.nn.sigmoid(gx[:, :H] + gh[:, :H])
        z = jax.nn.sigmoid(gx[:, H:2 * H] + gh[:, H:2 * H])
        n = jnp.tanh(gx[:, 2 * H:] + r * gh[:, 2 * H:])
        h_new = (1.0 - z) * n + z * h0
        seq_ref[pl.ds(row, NB), :] = h_new
        return h_new

    def step1(t, h1):
        row = t * NB
        gh = jnp.dot(h1, w1hh, preferred_element_type=jnp.float32) + b1hh
        gx = gx1_ref[pl.ds(row, NB), :]
        r = jax.nn.sigmoid(gx[:, :H] + gh[:, :H])
        z = jax.nn.sigmoid(gx[:, H:2 * H] + gh[:, H:2 * H])
        n = jnp.tanh(gx[:, 2 * H:] + r * gh[:, 2 * H:])
        return (1.0 - z) * n + z * h1

    def step_both(t, carry):
        h0, h1 = carry
        return step0(t, h0), step1(t, h1)

    @pl.when(c == 0)
    def _():
        hout_ref[0] = lax.fori_loop(0, Tt, step0, hout_ref[0], unroll=unroll)

    @pl.when(jnp.logical_and(c > 0, c < nc))
    def _():
        h0f, h1f = lax.fori_loop(0, Tt, step_both,
                                 (hout_ref[0], hout_ref[1]), unroll=unroll)
        hout_ref[0] = h0f
        hout_ref[1] = h1f

    @pl.when(c == nc)
    def _():
        h1f = lax.fori_loop(0, Tt, step1, hout_ref[1], unroll=unroll)
        hout_ref[1] = h1f
        h_last = jnp.maximum(h1f, 0.0)
        out_ref[...] = (jnp.dot(h_last, wfc_ref[...],
                                preferred_element_type=jnp.float32)
                        + bfc_ref[...])


def _cat_w(w):
    return jnp.concatenate([w[0], w[1], w[2]], axis=1)


def _cat_b(b):
    return jnp.concatenate([b[0], b[1], b[2]], axis=1)


@jax.jit
def _forward(x, h, fc_w, fc_b,
             l0_wih, l0_whh, l0_bih, l0_bhh,
             l1_wih, l1_whh, l1_bih, l1_bhh):
    B, D, T = x.shape
    L, _, H = h.shape
    C = fc_w.shape[1]

    x_btd = jnp.reshape(x, (B, T, D))

    Bp = _round_up(B, 16)
    if Bp != B:
        x_btd = jnp.pad(x_btd, ((0, Bp - B), (0, 0), (0, 0)))
        h = jnp.pad(h, ((0, 0), (0, Bp - B), (0, 0)))
    NB = Bp // 2

    x_t = jnp.transpose(x_btd, (1, 0, 2))

    Tt = _pick_time_chunk(T)
    nc = T // Tt
    unroll = True if Tt <= 16 else (8 if Tt % 8 == 0 else 1)

    w0ih, w1ih = _cat_w(l0_wih), _cat_w(l1_wih)
    w0hh, w1hh = _cat_w(l0_whh), _cat_w(l1_whh)
    b0ih, b1ih = _cat_b(l0_bih), _cat_b(l1_bih)
    b0hh, b1hh = _cat_b(l0_bhh), _cat_b(l1_bhh)

    def const2(b, c):
        return (0, 0)

    in_specs = [
        pl.BlockSpec((Tt, NB, D),
                     lambda b, c: (jnp.minimum(c, nc - 1), b, 0)),
        pl.BlockSpec((L, NB, H), lambda b, c: (0, b, 0)),
        pl.BlockSpec((D, 3 * H), const2),
        pl.BlockSpec((H, 3 * H), const2),
        pl.BlockSpec((1, 3 * H), const2),
        pl.BlockSpec((1, 3 * H), const2),
        pl.BlockSpec((H, 3 * H), const2),
        pl.BlockSpec((H, 3 * H), const2),
        pl.BlockSpec((1, 3 * H), const2),
        pl.BlockSpec((1, 3 * H), const2),
        pl.BlockSpec((H, C), const2),
        pl.BlockSpec((1, C), const2),
    ]
    out_specs = [
        pl.BlockSpec((NB, C), lambda b, c: (b, 0)),
        pl.BlockSpec((L, NB, H), lambda b, c: (0, b, 0)),
    ]

    out, h_new = pl.pallas_call(
        functools.partial(_gru_body, Tt=Tt, H=H, nc=nc, unroll=unroll),
        out_shape=(jax.ShapeDtypeStruct((Bp, C), jnp.float32),
                   jax.ShapeDtypeStruct((L, Bp, H), jnp.float32)),
        grid_spec=pltpu.PrefetchScalarGridSpec(
            num_scalar_prefetch=0,
            grid=(2, nc + 1),
            in_specs=in_specs,
            out_specs=out_specs,
            scratch_shapes=[pltpu.VMEM((Tt * NB, H), jnp.float32),
                            pltpu.VMEM((Tt * NB, 3 * H), jnp.float32),
                            pltpu.VMEM((Tt * NB, 3 * H), jnp.float32)],
        ),
        compiler_params=pltpu.CompilerParams(
            dimension_semantics=("parallel", "arbitrary")),
    )(x_t, h, w0ih, w0hh, b0ih, b0hh, w1ih, w1hh, b1ih, b1hh, fc_w, fc_b)

    return out[:B], h_new[:, :B]


def kernel(x, h, fc_w, fc_b,
           layer0_wih, layer0_whh, layer0_bih, layer0_bhh,
           layer1_wih, layer1_whh, layer1_bih, layer1_bhh):
    return _forward(x, h, fc_w, fc_b,
                    layer0_wih, layer0_whh, layer0_bih, layer0_bhh,
                    layer1_wih, layer1_whh, layer1_bih, layer1_bhh)


# speedup vs baseline: 1.0955x; 1.0381x over previous
import functools

import jax
import jax.numpy as jnp
from jax import lax
from jax.experimental import pallas as pl
from jax.experimental.pallas import tpu as pltpu


def _round_up(v, m):
    return ((v + m - 1) // m) * m


def _pick_time_chunk(T):
    if T <= 64:
        return T
    for tt in (64, 32, 16, 8):
        if T % tt == 0:
            return tt
    return T


def _gru_body(x_ref, h0_ref, wih0_ref, whh0_ref, bih0_ref, bhh0_ref,
              wih1_ref, whh1_ref, bih1_ref, bhh1_ref, wfc_ref, bfc_ref,
              out_ref, hout_ref, seq_ref, gx0_ref, gx1_ref,
              *, Tt, H, nc, unroll):
    c = pl.program_id(1)
    NB = h0_ref.shape[1]
    D = x_ref.shape[-1]

    @pl.when(c == 0)
    def _():
        hout_ref[...] = h0_ref[...]

    @pl.when(c > 0)
    def _():
        gx1_ref[...] = (jnp.dot(seq_ref[...], wih1_ref[...],
                                preferred_element_type=jnp.float32)
                        + bih1_ref[...])

    @pl.when(c < nc)
    def _():
        xin = x_ref[...].reshape(Tt * NB, D)
        gx0_ref[...] = (jnp.dot(xin, wih0_ref[...],
                                preferred_element_type=jnp.float32)
                        + bih0_ref[...])

    w0hh = whh0_ref[...]
    b0hh = bhh0_ref[...]
    w1hh = whh1_ref[...]
    b1hh = bhh1_ref[...]

    def step0(t, h0):
        row = t * NB
        gh = jnp.dot(h0, w0hh, preferred_element_type=jnp.float32) + b0hh
        gx = gx0_ref[pl.ds(row, NB), :]
        r = jax.nn.sigmoid(gx[:, :H] + gh[:, :H])
        z = jax.nn.sigmoid(gx[:, H:2 * H] + gh[:, H:2 * H])
        n = jnp.tanh(gx[:, 2 * H:] + r * gh[:, 2 * H:])
        h_new = (1.0 - z) * n + z * h0
        seq_ref[pl.ds(row, NB), :] = h_new
        return h_new

    def step1(t, h1):
        row = t * NB
        gh = jnp.dot(h1, w1hh, preferred_element_type=jnp.float32) + b1hh
        gx = gx1_ref[pl.ds(row, NB), :]
        r = jax.nn.sigmoid(gx[:, :H] + gh[:, :H])
        z = jax.nn.sigmoid(gx[:, H:2 * H] + gh[:, H:2 * H])
        n = jnp.tanh(gx[:, 2 * H:] + r * gh[:, 2 * H:])
        return (1.0 - z) * n + z * h1

    def step_both(t, carry):
        h0, h1 = carry
        return step0(t, h0), step1(t, h1)

    @pl.when(c == 0)
    def _():
        hout_ref[0] = lax.fori_loop(0, Tt, step0, hout_ref[0], unroll=unroll)

    @pl.when(jnp.logical_and(c > 0, c < nc))
    def _():
        h0f, h1f = lax.fori_loop(0, Tt, step_both,
                                 (hout_ref[0], hout_ref[1]), unroll=unroll)
        hout_ref[0] = h0f
        hout_ref[1] = h1f

    @pl.when(c == nc)
    def _():
        h1f = lax.fori_loop(0, Tt, step1, hout_ref[1], unroll=unroll)
        hout_ref[1] = h1f
        h_last = jnp.maximum(h1f, 0.0)
        out_ref[...] = (jnp.dot(h_last, wfc_ref[...],
                                preferred_element_type=jnp.float32)
                        + bfc_ref[...])


def _cat_w(w):
    return jnp.concatenate([w[0], w[1], w[2]], axis=1)


def _cat_b(b):
    return jnp.concatenate([b[0], b[1], b[2]], axis=1)


@jax.jit
def _forward(x, h, fc_w, fc_b,
             l0_wih, l0_whh, l0_bih, l0_bhh,
             l1_wih, l1_whh, l1_bih, l1_bhh):
    B, D, T = x.shape
    L, _, H = h.shape
    C = fc_w.shape[1]

    x_btd = jnp.reshape(x, (B, T, D))

    Bp = _round_up(B, 16)
    if Bp != B:
        x_btd = jnp.pad(x_btd, ((0, Bp - B), (0, 0), (0, 0)))
        h = jnp.pad(h, ((0, 0), (0, Bp - B), (0, 0)))
    NB = Bp // 2

    x_t = jnp.transpose(x_btd, (1, 0, 2))

    Tt = _pick_time_chunk(T)
    nc = T // Tt
    unroll = True if Tt <= 16 else (16 if Tt % 16 == 0 else 1)

    w0ih, w1ih = _cat_w(l0_wih), _cat_w(l1_wih)
    w0hh, w1hh = _cat_w(l0_whh), _cat_w(l1_whh)
    b0ih, b1ih = _cat_b(l0_bih), _cat_b(l1_bih)
    b0hh, b1hh = _cat_b(l0_bhh), _cat_b(l1_bhh)

    def const2(b, c):
        return (0, 0)

    in_specs = [
        pl.BlockSpec((Tt, NB, D),
                     lambda b, c: (jnp.minimum(c, nc - 1), b, 0)),
        pl.BlockSpec((L, NB, H), lambda b, c: (0, b, 0)),
        pl.BlockSpec((D, 3 * H), const2),
        pl.BlockSpec((H, 3 * H), const2),
        pl.BlockSpec((1, 3 * H), const2),
        pl.BlockSpec((1, 3 * H), const2),
        pl.BlockSpec((H, 3 * H), const2),
        pl.BlockSpec((H, 3 * H), const2),
        pl.BlockSpec((1, 3 * H), const2),
        pl.BlockSpec((1, 3 * H), const2),
        pl.BlockSpec((H, C), const2),
        pl.BlockSpec((1, C), const2),
    ]
    out_specs = [
        pl.BlockSpec((NB, C), lambda b, c: (b, 0)),
        pl.BlockSpec((L, NB, H), lambda b, c: (0, b, 0)),
    ]

    out, h_new = pl.pallas_call(
        functools.partial(_gru_body, Tt=Tt, H=H, nc=nc, unroll=unroll),
        out_shape=(jax.ShapeDtypeStruct((Bp, C), jnp.float32),
                   jax.ShapeDtypeStruct((L, Bp, H), jnp.float32)),
        grid_spec=pltpu.PrefetchScalarGridSpec(
            num_scalar_prefetch=0,
            grid=(2, nc + 1),
            in_specs=in_specs,
            out_specs=out_specs,
            scratch_shapes=[pltpu.VMEM((Tt * NB, H), jnp.float32),
                            pltpu.VMEM((Tt * NB, 3 * H), jnp.float32),
                            pltpu.VMEM((Tt * NB, 3 * H), jnp.float32)],
        ),
        compiler_params=pltpu.CompilerParams(
            dimension_semantics=("parallel", "arbitrary")),
    )(x_t, h, w0ih, w0hh, b0ih, b0hh, w1ih, w1hh, b1ih, b1hh, fc_w, fc_b)

    return out[:B], h_new[:, :B]


def kernel(x, h, fc_w, fc_b,
           layer0_wih, layer0_whh, layer0_bih, layer0_bhh,
           layer1_wih, layer1_whh, layer1_bih, layer1_bhh):
    return _forward(x, h, fc_w, fc_b,
                    layer0_wih, layer0_whh, layer0_bih, layer0_bhh,
                    layer1_wih, layer1_whh, layer1_bih, layer1_bhh)


# speedup vs baseline: 1.1288x; 1.0304x over previous
import functools

import jax
import jax.numpy as jnp
from jax import lax
from jax.experimental import pallas as pl
from jax.experimental.pallas import tpu as pltpu


def _round_up(v, m):
    return ((v + m - 1) // m) * m


def _pick_time_chunk(T):
    if T <= 64:
        return T
    for tt in (64, 32, 16, 8):
        if T % tt == 0:
            return tt
    return T


def _gru_body(x_ref, h0_ref, wih0_ref, whh0_ref, bih0_ref, bhh0_ref,
              wih1_ref, whh1_ref, bih1_ref, bhh1_ref, wfc_ref, bfc_ref,
              out_ref, hout_ref, seq_ref, gx0_ref, gx1_ref,
              *, Tt, H, nc, unroll):
    c = pl.program_id(1)
    NB = h0_ref.shape[1]
    D = x_ref.shape[-1]

    @pl.when(c == 0)
    def _():
        hout_ref[...] = h0_ref[...]

    @pl.when(c > 0)
    def _():
        gx1_ref[...] = (jnp.dot(seq_ref[...], wih1_ref[...],
                                preferred_element_type=jnp.float32)
                        + bih1_ref[...])

    @pl.when(c < nc)
    def _():
        xin = x_ref[...].reshape(Tt * NB, D)
        gx0_ref[...] = (jnp.dot(xin, wih0_ref[...],
                                preferred_element_type=jnp.float32)
                        + bih0_ref[...])

    w0hh = whh0_ref[...]
    b0hh = bhh0_ref[...]
    w1hh = whh1_ref[...]
    b1hh = bhh1_ref[...]

    def step0(t, h0):
        row = t * NB
        gh = jnp.dot(h0, w0hh, preferred_element_type=jnp.float32) + b0hh
        gx = gx0_ref[pl.ds(row, NB), :]
        r = jax.nn.sigmoid(gx[:, :H] + gh[:, :H])
        z = jax.nn.sigmoid(gx[:, H:2 * H] + gh[:, H:2 * H])
        n = jnp.tanh(gx[:, 2 * H:] + r * gh[:, 2 * H:])
        h_new = (1.0 - z) * n + z * h0
        seq_ref[pl.ds(row, NB), :] = h_new
        return h_new

    def step1(t, h1):
        row = t * NB
        gh = jnp.dot(h1, w1hh, preferred_element_type=jnp.float32) + b1hh
        gx = gx1_ref[pl.ds(row, NB), :]
        r = jax.nn.sigmoid(gx[:, :H] + gh[:, :H])
        z = jax.nn.sigmoid(gx[:, H:2 * H] + gh[:, H:2 * H])
        n = jnp.tanh(gx[:, 2 * H:] + r * gh[:, 2 * H:])
        return (1.0 - z) * n + z * h1

    def step_both(t, carry):
        h0, h1 = carry
        return step0(t, h0), step1(t, h1)

    @pl.when(c == 0)
    def _():
        hout_ref[0] = lax.fori_loop(0, Tt, step0, hout_ref[0], unroll=unroll)

    @pl.when(jnp.logical_and(c > 0, c < nc))
    def _():
        h0f, h1f = lax.fori_loop(0, Tt, step_both,
                                 (hout_ref[0], hout_ref[1]), unroll=unroll)
        hout_ref[0] = h0f
        hout_ref[1] = h1f

    @pl.when(c == nc)
    def _():
        h1f = lax.fori_loop(0, Tt, step1, hout_ref[1], unroll=unroll)
        hout_ref[1] = h1f
        h_last = jnp.maximum(h1f, 0.0)
        out_ref[...] = (jnp.dot(h_last, wfc_ref[...],
                                preferred_element_type=jnp.float32)
                        + bfc_ref[...])


def _cat_w(w):
    return jnp.concatenate([w[0], w[1], w[2]], axis=1)


def _cat_b(b):
    return jnp.concatenate([b[0], b[1], b[2]], axis=1)


@jax.jit
def _forward(x, h, fc_w, fc_b,
             l0_wih, l0_whh, l0_bih, l0_bhh,
             l1_wih, l1_whh, l1_bih, l1_bhh):
    B, D, T = x.shape
    L, _, H = h.shape
    C = fc_w.shape[1]

    x_btd = jnp.reshape(x, (B, T, D))

    Bp = _round_up(B, 16)
    if Bp != B:
        x_btd = jnp.pad(x_btd, ((0, Bp - B), (0, 0), (0, 0)))
        h = jnp.pad(h, ((0, 0), (0, Bp - B), (0, 0)))
    NB = Bp // 2

    x_t = jnp.transpose(x_btd, (1, 0, 2))

    Tt = _pick_time_chunk(T)
    nc = T // Tt
    unroll = True if Tt <= 64 else (16 if Tt % 16 == 0 else 1)

    w0ih, w1ih = _cat_w(l0_wih), _cat_w(l1_wih)
    w0hh, w1hh = _cat_w(l0_whh), _cat_w(l1_whh)
    b0ih, b1ih = _cat_b(l0_bih), _cat_b(l1_bih)
    b0hh, b1hh = _cat_b(l0_bhh), _cat_b(l1_bhh)

    def const2(b, c):
        return (0, 0)

    in_specs = [
        pl.BlockSpec((Tt, NB, D),
                     lambda b, c: (jnp.minimum(c, nc - 1), b, 0)),
        pl.BlockSpec((L, NB, H), lambda b, c: (0, b, 0)),
        pl.BlockSpec((D, 3 * H), const2),
        pl.BlockSpec((H, 3 * H), const2),
        pl.BlockSpec((1, 3 * H), const2),
        pl.BlockSpec((1, 3 * H), const2),
        pl.BlockSpec((H, 3 * H), const2),
        pl.BlockSpec((H, 3 * H), const2),
        pl.BlockSpec((1, 3 * H), const2),
        pl.BlockSpec((1, 3 * H), const2),
        pl.BlockSpec((H, C), const2),
        pl.BlockSpec((1, C), const2),
    ]
    out_specs = [
        pl.BlockSpec((NB, C), lambda b, c: (b, 0)),
        pl.BlockSpec((L, NB, H), lambda b, c: (0, b, 0)),
    ]

    out, h_new = pl.pallas_call(
        functools.partial(_gru_body, Tt=Tt, H=H, nc=nc, unroll=unroll),
        out_shape=(jax.ShapeDtypeStruct((Bp, C), jnp.float32),
                   jax.ShapeDtypeStruct((L, Bp, H), jnp.float32)),
        grid_spec=pltpu.PrefetchScalarGridSpec(
            num_scalar_prefetch=0,
            grid=(2, nc + 1),
            in_specs=in_specs,
            out_specs=out_specs,
            scratch_shapes=[pltpu.VMEM((Tt * NB, H), jnp.float32),
                            pltpu.VMEM((Tt * NB, 3 * H), jnp.float32),
                            pltpu.VMEM((Tt * NB, 3 * H), jnp.float32)],
        ),
        compiler_params=pltpu.CompilerParams(
            dimension_semantics=("parallel", "arbitrary")),
    )(x_t, h, w0ih, w0hh, b0ih, b0hh, w1ih, w1hh, b1ih, b1hh, fc_w, fc_b)

    return out[:B], h_new[:, :B]


def kernel(x, h, fc_w, fc_b,
           layer0_wih, layer0_whh, layer0_bih, layer0_bhh,
           layer1_wih, layer1_whh, layer1_bih, layer1_bhh):
    return _forward(x, h, fc_w, fc_b,
                    layer0_wih, layer0_whh, layer0_bih, layer0_bhh,
                    layer1_wih, layer1_whh, layer1_bih, layer1_bhh)
